# Initial kernel scaffold; baseline (speedup 1.0000x reference)
import jax
import jax.numpy as jnp
from jax.experimental import pallas as pl


def kernel(x0, x1, W0, b0, W1, b1, a_src, a_dst, Wout, bout, edge_index):
    raise NotImplementedError("write your pallas kernel here")


# speedup vs baseline: 1.1630x; 1.1630x over previous
import jax
import jax.numpy as jnp
from jax.experimental import pallas as pl

ALPHA = 0.2
N_NODES = 10000
N_HEADS = 4


def _leaky(x):
    return jnp.where(x >= 0, x, ALPHA * x)


def _proj_body(x_ref, W_ref, b_ref, A_ref, h_ref, e_ref):
    h = _leaky(
        jnp.dot(x_ref[...], W_ref[...], preferred_element_type=jnp.float32)
        + b_ref[...]
    )
    h_ref[...] = h
    e_ref[...] = jnp.dot(h, A_ref[...], preferred_element_type=jnp.float32)


def _proj(x, W, b, A, blk=1000):
    R, D_in = x.shape
    D_h = W.shape[1]
    return pl.pallas_call(
        _proj_body,
        grid=(R // blk,),
        in_specs=[
            pl.BlockSpec((blk, D_in), lambda i: (i, 0)),
            pl.BlockSpec((D_in, D_h), lambda i: (0, 0)),
            pl.BlockSpec((1, D_h), lambda i: (0, 0)),
            pl.BlockSpec((D_h, 8), lambda i: (0, 0)),
        ],
        out_specs=[
            pl.BlockSpec((blk, D_h), lambda i: (i, 0)),
            pl.BlockSpec((blk, 8), lambda i: (i, 0)),
        ],
        out_shape=[
            jax.ShapeDtypeStruct((R, D_h), jnp.float32),
            jax.ShapeDtypeStruct((R, 8), jnp.float32),
        ],
    )(x, W, b, A)


def _out_body(f_ref, W_ref, b_ref, o_ref):
    o = (
        jnp.dot(f_ref[...], W_ref[...], preferred_element_type=jnp.float32)
        + b_ref[...]
    )
    nrm = jnp.sqrt(jnp.sum(o * o, axis=1, keepdims=True))
    o_ref[...] = o / jnp.maximum(nrm, 1e-12)


def _out_proj(feat, Wout, bout, blk=1000):
    R, D_h = feat.shape
    D_o = Wout.shape[1]
    return pl.pallas_call(
        _out_body,
        grid=(R // blk,),
        in_specs=[
            pl.BlockSpec((blk, D_h), lambda i: (i, 0)),
            pl.BlockSpec((D_h, D_o), lambda i: (0, 0)),
            pl.BlockSpec((1, D_o), lambda i: (0, 0)),
        ],
        out_specs=pl.BlockSpec((blk, D_o), lambda i: (i, 0)),
        out_shape=jax.ShapeDtypeStruct((R, D_o), jnp.float32),
    )(feat, Wout, bout)


def kernel(x0, x1, W0, b0, W1, b1, a_src, a_dst, Wout, bout, edge_index):
    A = jnp.concatenate([a_src.T, a_dst.T], axis=1)
    h0, e0 = _proj(x0, W0, b0.reshape(1, -1), A)
    h1, e1 = _proj(x1, W1, b1.reshape(1, -1), A)
    h = jnp.concatenate([h0, h1], axis=0)
    ga = jnp.concatenate([e0, e1], axis=0)
    gs = ga[:, :N_HEADS]
    gd = ga[:, N_HEADS:]

    src = edge_index[0]
    dst = edge_index[1]
    e = _leaky(jnp.take(gs, src, axis=0) + jnp.take(gd, dst, axis=0))
    w = jnp.exp(e)
    denom = jax.ops.segment_sum(w, dst, num_segments=N_NODES)
    hs = jnp.take(h, src, axis=0)
    heads = []
    for k in range(N_HEADS):
        m = jax.ops.segment_max(w[:, k : k + 1] * hs, dst, num_segments=N_NODES)
        dk = denom[:, k : k + 1]
        heads.append(jnp.where(dk > 0, m / jnp.maximum(dk, 1e-30), 0.0))
    feat = jnp.mean(jnp.stack(heads, axis=0), axis=0)
    return _out_proj(feat, Wout, bout.reshape(1, -1))
